# baseline (speedup 1.0000x reference)
import functools

import jax
import jax.numpy as jnp
from jax import lax
from jax.experimental import pallas as pl
from jax.experimental.pallas import tpu as pltpu
from jax.experimental.pallas import tpu_sc as plsc

N = 2048
E = 65536
R = 4
NR = N * R
NC = 2
NS = 16
EPW = E // (NC * NS)

_MESH = dict(core_axis_name="c", subcore_axis_name="s", num_cores=NC,
             num_subcores=NS)


def _sc_count(keyrows, ones128, zeros16):
    nb = EPW // 128

    @functools.partial(
        pl.kernel,
        out_type=jax.ShapeDtypeStruct((NC * NR, 16), jnp.float32),
        mesh=plsc.VectorSubcoreMesh(**_MESH),
        compiler_params=pltpu.CompilerParams(use_tc_tiling_on_sc=False),
        scratch_types=[
            pltpu.VMEM((nb, 128), jnp.int32),
            pltpu.VMEM((128, 16), jnp.float32),
            pltpu.SemaphoreType.DMA,
            pltpu.SemaphoreType.DMA,
            pltpu.VMEM_SHARED((NR, 16), jnp.float32),
        ],
    )
    def k(key_hbm, ones_hbm, zeros_hbm, out_hbm, key_v, ones_v, c0, c1,
          shared):
        cid = lax.axis_index("c")
        sid = lax.axis_index("s")
        rows = NR // NS
        csem = (c0, c1)
        pltpu.sync_copy(ones_hbm, ones_v)
        pltpu.sync_copy(zeros_hbm, shared.at[pl.ds(sid * rows, rows)])
        pltpu.sync_copy(key_hbm.at[pl.ds((cid * NS + sid) * nb, nb)],
                        key_v)
        plsc.subcore_barrier()

        def body(g, _):
            for b in range(2):
                j = g * 2 + b

                @pl.when(j >= 2)
                def _():
                    pltpu.make_async_copy(ones_v,
                                          shared.at[key_v.at[j - 2]],
                                          csem[b]).wait()

                pltpu.async_copy(ones_v, shared.at[key_v.at[j]], csem[b],
                                 add=True)
            return 0

        lax.fori_loop(0, nb // 2, body, 0)
        for tail in (nb - 2, nb - 1):
            pltpu.make_async_copy(ones_v, shared.at[key_v.at[tail]],
                                  csem[tail % 2]).wait()
        plsc.subcore_barrier()
        pltpu.sync_copy(shared.at[pl.ds(sid * rows, rows)],
                        out_hbm.at[pl.ds(cid * NR + sid * rows, rows)])

    return k(keyrows, ones128, zeros16).reshape(NC, R, N, 16)


def _sc_bucket_wide(table, idxrows, keyrows, zeros64):
    eps = E // NS
    nb = eps // 128

    @functools.partial(
        pl.kernel,
        out_type=jax.ShapeDtypeStruct((NC * NR, 128), jnp.float32),
        mesh=plsc.VectorSubcoreMesh(**_MESH),
        scratch_types=[
            pltpu.VMEM((nb, 128), jnp.int32),
            pltpu.VMEM((nb, 128), jnp.int32),
            pltpu.VMEM((2, 128, 128), jnp.float32),
            pltpu.VMEM((64, 128), jnp.float32),
            pltpu.SemaphoreType.DMA,
            pltpu.SemaphoreType.DMA,
            pltpu.VMEM_SHARED((NR, 128), jnp.float32),
        ],
    )
    def k(tab_hbm, idx_hbm, key_hbm, zeros_hbm, out_hbm, idx_v, key_v,
          rows_v, zero_v, sem0, sem1, shared):
        cid = lax.axis_index("c")
        sid = lax.axis_index("s")
        rows = NR // NS
        sems = (sem0, sem1)
        pltpu.sync_copy(
            idx_hbm.at[pl.ds(cid * (E // 128) + sid * nb, nb)], idx_v)
        pltpu.sync_copy(key_hbm.at[pl.ds(sid * nb, nb)], key_v)
        pltpu.async_copy(tab_hbm.at[idx_v.at[0]], rows_v.at[0], sem0)
        pltpu.sync_copy(zeros_hbm, zero_v)
        for z in range(rows // 64):
            pltpu.sync_copy(zero_v,
                            shared.at[pl.ds(sid * rows + z * 64, 64)])
        plsc.subcore_barrier()


        def body(g, _):
            for b in range(2):
                j = g * 2 + b

                @pl.when(j + 1 < nb)
                def _():
                    pltpu.async_copy(tab_hbm.at[idx_v.at[j + 1]],
                                     rows_v.at[1 - b], sems[1 - b])

                pltpu.make_async_copy(tab_hbm.at[idx_v.at[j]],
                                      rows_v.at[b], sems[b]).wait()
                pltpu.sync_copy(rows_v.at[b], shared.at[key_v.at[j]],
                                add=True)
            return 0

        lax.fori_loop(0, nb // 2, body, 0)
        plsc.subcore_barrier()
        pltpu.sync_copy(shared.at[pl.ds(sid * rows, rows)],
                        out_hbm.at[pl.ds(cid * NR + sid * rows, rows)])

    return k(table, idxrows, keyrows, zeros64).reshape(NC, R, N, 128)


def _sc_bucket_narrow(table, gidxrows, keyrows, zeros16):
    nb = EPW // 128

    @functools.partial(
        pl.kernel,
        out_type=jax.ShapeDtypeStruct((NC * NR, 16), jnp.float32),
        mesh=plsc.VectorSubcoreMesh(**_MESH),
        compiler_params=pltpu.CompilerParams(use_tc_tiling_on_sc=False),
        scratch_types=[
            pltpu.VMEM((nb, 128), jnp.int32),
            pltpu.VMEM((nb, 128), jnp.int32),
            pltpu.VMEM((2, 128, 16), jnp.float32),
            pltpu.SemaphoreType.DMA,
            pltpu.SemaphoreType.DMA,
            pltpu.SemaphoreType.DMA,
            pltpu.SemaphoreType.DMA,
            pltpu.VMEM_SHARED((NR, 16), jnp.float32),
        ],
    )
    def k(tab_hbm, idx_hbm, key_hbm, zeros_hbm, out_hbm, idx_v, key_v,
          rows_v, g0, g1, s0, s1, shared):
        cid = lax.axis_index("c")
        sid = lax.axis_index("s")
        rows = NR // NS
        gsem = (g0, g1)
        ssem = (s0, s1)
        base = (cid * NS + sid) * nb
        pltpu.sync_copy(idx_hbm.at[pl.ds(base, nb)], idx_v)
        pltpu.sync_copy(key_hbm.at[pl.ds(base, nb)], key_v)
        pltpu.async_copy(tab_hbm.at[idx_v.at[0]], rows_v.at[0], g0)
        pltpu.sync_copy(zeros_hbm, shared.at[pl.ds(sid * rows, rows)])
        plsc.subcore_barrier()

        def body(g, _):
            for b in range(2):
                j = g * 2 + b
                pltpu.make_async_copy(tab_hbm.at[idx_v.at[j]],
                                      rows_v.at[b], gsem[b]).wait()
                pltpu.async_copy(rows_v.at[b], shared.at[key_v.at[j]],
                                 ssem[b], add=True)

                @pl.when(j + 1 < nb)
                def _():
                    @pl.when(j >= 1)
                    def _():
                        pltpu.make_async_copy(
                            rows_v.at[1 - b], shared.at[key_v.at[j - 1]],
                            ssem[1 - b]).wait()

                    pltpu.async_copy(tab_hbm.at[idx_v.at[j + 1]],
                                     rows_v.at[1 - b], gsem[1 - b])
            return 0

        lax.fori_loop(0, nb // 2, body, 0)
        for tail in (nb - 2, nb - 1):
            pltpu.make_async_copy(rows_v.at[tail % 2],
                                  shared.at[key_v.at[tail]],
                                  ssem[tail % 2]).wait()
        plsc.subcore_barrier()
        pltpu.sync_copy(shared.at[pl.ds(sid * rows, rows)],
                        out_hbm.at[pl.ds(cid * NR + sid * rows, rows)])

    return k(table, gidxrows, keyrows, zeros16).reshape(NC, R, N, 16)


_GRID_B = 8
_BN = N // _GRID_B


def _tc_wide_body(bkt_ref, cnt_ref, hp_ref, w_ref, root_ref, b_ref, o_ref):
    hp = jnp.concatenate([hp_ref[0], hp_ref[1]], axis=1)
    agg = jax.lax.dot(hp, root_ref[...],
                      preferred_element_type=jnp.float32) + b_ref[...]
    for r in range(R):
        tot = cnt_ref[0, r, :, 0:1] + cnt_ref[1, r, :, 0:1]
        inv = 1.0 / jnp.maximum(tot, 1.0)
        for c in range(2):
            agg += jax.lax.dot(bkt_ref[c, r] * inv, w_ref[c, r],
                               preferred_element_type=jnp.float32)
    out = jnp.maximum(agg, 0.0)
    o_ref[0] = out[:, :128]
    o_ref[1] = out[:, 128:]


def _tc_wide(buckets, cnt4, hprev, Wstack, root, b):
    return pl.pallas_call(
        _tc_wide_body,
        grid=(_GRID_B,),
        in_specs=[
            pl.BlockSpec((2, R, _BN, 128), lambda i: (0, 0, i, 0)),
            pl.BlockSpec((2, R, _BN, 16), lambda i: (0, 0, i, 0)),
            pl.BlockSpec((2, _BN, 128), lambda i: (0, i, 0)),
            pl.BlockSpec((2, R, 128, 256), lambda i: (0, 0, 0, 0)),
            pl.BlockSpec((256, 256), lambda i: (0, 0)),
            pl.BlockSpec((1, 256), lambda i: (0, 0)),
        ],
        out_specs=pl.BlockSpec((2, _BN, 128), lambda i: (0, i, 0)),
        out_shape=jax.ShapeDtypeStruct((2, N, 128), jnp.float32),
    )(buckets, cnt4, hprev, Wstack, root, b)


def _tc_msg_body(hp_ref, w_ref, o_ref):
    hp = jnp.concatenate([hp_ref[0], hp_ref[1]], axis=1)
    for r in range(R):
        o_ref[r] = jax.lax.dot(hp, w_ref[r],
                               preferred_element_type=jnp.float32)


def _tc_msg(hprev, Wpad):
    return pl.pallas_call(
        _tc_msg_body,
        grid=(_GRID_B,),
        in_specs=[
            pl.BlockSpec((2, _BN, 128), lambda i: (0, i, 0)),
            pl.BlockSpec((R, 256, 16), lambda i: (0, 0, 0)),
        ],
        out_specs=pl.BlockSpec((R, _BN, 16), lambda i: (0, i, 0)),
        out_shape=jax.ShapeDtypeStruct((R, N, 16), jnp.float32),
    )(hprev, Wpad)


def _tc_narrow_body(wide_h, bkt_ref, cnt_ref, hp_ref, m_ref, root_ref,
                    b_ref, o_ref):
    if wide_h:
        hp = jnp.concatenate([hp_ref[0], hp_ref[1]], axis=1)
    else:
        hp = hp_ref[...]
    agg = jax.lax.dot(hp, root_ref[...],
                      preferred_element_type=jnp.float32) + b_ref[...]
    for r in range(R):
        tot = cnt_ref[0, r, :, 0:1] + cnt_ref[1, r, :, 0:1]
        inv = 1.0 / jnp.maximum(tot, 1.0)
        agg += jax.lax.dot((bkt_ref[0, r] + bkt_ref[1, r]) * inv, m_ref[r],
                           preferred_element_type=jnp.float32)
    o_ref[...] = jnp.maximum(agg, 0.0)


def _tc_narrow(buckets, cnt4, hprev, mix, rootpad, bpad, wide_h):
    hspec = (pl.BlockSpec((2, _BN, 128), lambda i: (0, i, 0)) if wide_h
             else pl.BlockSpec((_BN, 16), lambda i: (i, 0)))
    rootdim = 256 if wide_h else 16
    return pl.pallas_call(
        functools.partial(_tc_narrow_body, wide_h),
        grid=(_GRID_B,),
        in_specs=[
            pl.BlockSpec((2, R, _BN, 16), lambda i: (0, 0, i, 0)),
            pl.BlockSpec((2, R, _BN, 16), lambda i: (0, 0, i, 0)),
            hspec,
            pl.BlockSpec((R, 16, 16), lambda i: (0, 0, 0)),
            pl.BlockSpec((rootdim, 16), lambda i: (0, 0)),
            pl.BlockSpec((1, 16), lambda i: (0, 0)),
        ],
        out_specs=pl.BlockSpec((_BN, 16), lambda i: (i, 0)),
        out_shape=jax.ShapeDtypeStruct((N, 16), jnp.float32),
    )(buckets, cnt4, hprev, mix, rootpad, bpad)


def _matvec_body(act, v_ref, m_ref, b_ref, o_ref, acc_ref):
    kk = pl.program_id(1)
    nk = pl.num_programs(1)

    @pl.when(kk == 0)
    def _():
        acc_ref[...] = jnp.zeros_like(acc_ref)

    acc_ref[...] += jax.lax.dot(v_ref[...], m_ref[...],
                                preferred_element_type=jnp.float32)

    @pl.when(kk == nk - 1)
    def _():
        o_ref[...] = act(acc_ref[...] + b_ref[...])


def _matvec(v, M, b, act, bn=512, bk=2048):
    K, Nout = M.shape
    out = pl.pallas_call(
        functools.partial(_matvec_body, act),
        grid=(Nout // bn, K // bk),
        in_specs=[
            pl.BlockSpec((1, bk), lambda j, k: (0, k)),
            pl.BlockSpec((bk, bn), lambda j, k: (k, j)),
            pl.BlockSpec((1, bn), lambda j, k: (0, j)),
        ],
        out_specs=pl.BlockSpec((1, bn), lambda j, k: (0, j)),
        out_shape=jax.ShapeDtypeStruct((1, Nout), jnp.float32),
        scratch_shapes=[pltpu.VMEM((1, bn), jnp.float32)],
    )(v.reshape(1, K), M, b.reshape(1, Nout))
    return out.reshape(Nout)


def kernel(x, edge_index, edge_type, W0, root0, b0, W1, root1, b1, W2,
           root2, b2, W3, root3, b3, W4, root4, b4, pw1, pb1, pw2, pb2,
           pw3, pb3):
    src, dst = edge_index[0], edge_index[1]
    key = edge_type * N + dst
    keyrows = key.reshape(E // 128, 128)
    src2 = jnp.concatenate([src, src + N]).reshape(2 * E // 128, 128)
    gidx3 = (edge_type * N + src).reshape(E // 128, 128)
    srcrows = src.reshape(E // 128, 128)
    ones128 = jnp.ones((128, 16), jnp.float32)
    zeros16 = jnp.zeros((NR // NS, 16), jnp.float32)
    zeros64 = jnp.zeros((64, 128), jnp.float32)

    cnt4 = _sc_count(keyrows, ones128, zeros16)

    def pad16(w):
        return jnp.pad(w, [(0, 0)] * (w.ndim - 1) + [(0, 16 - w.shape[-1])])

    h2 = x.reshape(N, 2, 128).transpose(1, 0, 2)
    for (W, root, b) in [(W0, root0, b0), (W1, root1, b1), (W2, root2, b2)]:
        buckets = _sc_bucket_wide(h2.reshape(2 * N, 128), src2, keyrows,
                                  zeros64)
        Wstack = jnp.stack([W[:, :128, :], W[:, 128:, :]])
        h2 = _tc_wide(buckets, cnt4, h2, Wstack, root, b.reshape(1, 256))

    am_tab = _tc_msg(h2, pad16(W3))
    b3kt = _sc_bucket_narrow(am_tab.reshape(R * N, 16), gidx3, keyrows,
                             zeros16)
    sum_r = jnp.broadcast_to(jnp.eye(16, dtype=jnp.float32), (R, 16, 16))
    h4 = _tc_narrow(b3kt, cnt4, h2, sum_r, pad16(root3),
                    pad16(b3).reshape(1, 16), wide_h=True)

    b4kt = _sc_bucket_narrow(h4, srcrows, keyrows, zeros16)
    W4mix = pad16(jnp.pad(W4, ((0, 0), (0, 13), (0, 0))))
    root4pad = pad16(jnp.pad(root4, ((0, 13), (0, 0))))
    h5 = _tc_narrow(b4kt, cnt4, h4, W4mix, root4pad,
                    pad16(b4).reshape(1, 16), wide_h=False)

    flat = h5[:, :3].reshape(-1)
    z = _matvec(flat, pw1, pb1, jnp.tanh, bn=2048)
    z = _matvec(z, pw2, pb2, jnp.tanh, bn=2048)
    z = _matvec(z, pw3, pb3, jax.nn.sigmoid, bn=2048)
    return z


# speedup vs baseline: 1.0225x; 1.0225x over previous
import functools

import jax
import jax.numpy as jnp
from jax import lax
from jax.experimental import pallas as pl
from jax.experimental.pallas import tpu as pltpu
from jax.experimental.pallas import tpu_sc as plsc

N = 2048
E = 65536
R = 4
NR = N * R
NC = 2
NS = 16
EPW = E // (NC * NS)

_MESH = dict(core_axis_name="c", subcore_axis_name="s", num_cores=NC,
             num_subcores=NS)


def _sc_count(keyrows, ones128, zeros16):
    nb = EPW // 128

    @functools.partial(
        pl.kernel,
        out_type=jax.ShapeDtypeStruct((NC * NR, 16), jnp.float32),
        mesh=plsc.VectorSubcoreMesh(**_MESH),
        compiler_params=pltpu.CompilerParams(use_tc_tiling_on_sc=False),
        scratch_types=[
            pltpu.VMEM((nb, 128), jnp.int32),
            pltpu.VMEM((128, 16), jnp.float32),
            pltpu.VMEM_SHARED((NR, 16), jnp.float32),
        ],
    )
    def k(key_hbm, ones_hbm, zeros_hbm, out_hbm, key_v, ones_v, shared):
        cid = lax.axis_index("c")
        sid = lax.axis_index("s")
        rows = NR // NS
        pltpu.sync_copy(ones_hbm, ones_v)
        pltpu.sync_copy(zeros_hbm, shared.at[pl.ds(sid * rows, rows)])
        pltpu.sync_copy(key_hbm.at[pl.ds((cid * NS + sid) * nb, nb)],
                        key_v)
        plsc.subcore_barrier()

        def body(g, _):
            pltpu.sync_copy(ones_v, shared.at[key_v.at[g]], add=True)
            return 0

        lax.fori_loop(0, nb, body, 0)
        plsc.subcore_barrier()
        pltpu.sync_copy(shared.at[pl.ds(sid * rows, rows)],
                        out_hbm.at[pl.ds(cid * NR + sid * rows, rows)])

    return k(keyrows, ones128, zeros16).reshape(NC, R, N, 16)


def _sc_bucket_wide(table, idxrows, keyrows, zeros64):
    eps = E // NS
    nb = eps // 128

    @functools.partial(
        pl.kernel,
        out_type=jax.ShapeDtypeStruct((NC * NR, 128), jnp.float32),
        mesh=plsc.VectorSubcoreMesh(**_MESH),
        scratch_types=[
            pltpu.VMEM((nb, 128), jnp.int32),
            pltpu.VMEM((nb, 128), jnp.int32),
            pltpu.VMEM((2, 128, 128), jnp.float32),
            pltpu.VMEM((64, 128), jnp.float32),
            pltpu.SemaphoreType.DMA,
            pltpu.SemaphoreType.DMA,
            pltpu.VMEM_SHARED((NR, 128), jnp.float32),
        ],
    )
    def k(tab_hbm, idx_hbm, key_hbm, zeros_hbm, out_hbm, idx_v, key_v,
          rows_v, zero_v, sem0, sem1, shared):
        cid = lax.axis_index("c")
        sid = lax.axis_index("s")
        rows = NR // NS
        sems = (sem0, sem1)
        pltpu.sync_copy(
            idx_hbm.at[pl.ds(cid * (E // 128) + sid * nb, nb)], idx_v)
        pltpu.sync_copy(key_hbm.at[pl.ds(sid * nb, nb)], key_v)
        pltpu.async_copy(tab_hbm.at[idx_v.at[0]], rows_v.at[0], sem0)
        pltpu.sync_copy(zeros_hbm, zero_v)
        for z in range(rows // 64):
            pltpu.sync_copy(zero_v,
                            shared.at[pl.ds(sid * rows + z * 64, 64)])
        plsc.subcore_barrier()


        def body(g, _):
            for b in range(2):
                j = g * 2 + b

                @pl.when(j + 1 < nb)
                def _():
                    pltpu.async_copy(tab_hbm.at[idx_v.at[j + 1]],
                                     rows_v.at[1 - b], sems[1 - b])

                pltpu.make_async_copy(tab_hbm.at[idx_v.at[j]],
                                      rows_v.at[b], sems[b]).wait()
                pltpu.sync_copy(rows_v.at[b], shared.at[key_v.at[j]],
                                add=True)
            return 0

        lax.fori_loop(0, nb // 2, body, 0)
        plsc.subcore_barrier()
        pltpu.sync_copy(shared.at[pl.ds(sid * rows, rows)],
                        out_hbm.at[pl.ds(cid * NR + sid * rows, rows)])

    return k(table, idxrows, keyrows, zeros64).reshape(NC, R, N, 128)


def _sc_bucket_narrow(table, gidxrows, keyrows, zeros16):
    nb = EPW // 128

    @functools.partial(
        pl.kernel,
        out_type=jax.ShapeDtypeStruct((NC * NR, 16), jnp.float32),
        mesh=plsc.VectorSubcoreMesh(**_MESH),
        compiler_params=pltpu.CompilerParams(use_tc_tiling_on_sc=False),
        scratch_types=[
            pltpu.VMEM((nb, 128), jnp.int32),
            pltpu.VMEM((nb, 128), jnp.int32),
            pltpu.VMEM((2, 128, 16), jnp.float32),
            pltpu.SemaphoreType.DMA,
            pltpu.SemaphoreType.DMA,
            pltpu.VMEM_SHARED((NR, 16), jnp.float32),
        ],
    )
    def k(tab_hbm, idx_hbm, key_hbm, zeros_hbm, out_hbm, idx_v, key_v,
          rows_v, sem0, sem1, shared):
        cid = lax.axis_index("c")
        sid = lax.axis_index("s")
        rows = NR // NS
        sems = (sem0, sem1)
        base = (cid * NS + sid) * nb
        pltpu.sync_copy(idx_hbm.at[pl.ds(base, nb)], idx_v)
        pltpu.sync_copy(key_hbm.at[pl.ds(base, nb)], key_v)
        pltpu.async_copy(tab_hbm.at[idx_v.at[0]], rows_v.at[0], sem0)
        pltpu.sync_copy(zeros_hbm, shared.at[pl.ds(sid * rows, rows)])
        plsc.subcore_barrier()

        def body(g, _):
            for b in range(2):
                j = g * 2 + b

                @pl.when(j + 1 < nb)
                def _():
                    pltpu.async_copy(tab_hbm.at[idx_v.at[j + 1]],
                                     rows_v.at[1 - b], sems[1 - b])

                pltpu.make_async_copy(tab_hbm.at[idx_v.at[j]],
                                      rows_v.at[b], sems[b]).wait()
                pltpu.sync_copy(rows_v.at[b], shared.at[key_v.at[j]],
                                add=True)
            return 0

        lax.fori_loop(0, nb // 2, body, 0)
        plsc.subcore_barrier()
        pltpu.sync_copy(shared.at[pl.ds(sid * rows, rows)],
                        out_hbm.at[pl.ds(cid * NR + sid * rows, rows)])

    return k(table, gidxrows, keyrows, zeros16).reshape(NC, R, N, 16)


_GRID_B = 8
_BN = N // _GRID_B


def _tc_wide_body(bkt_ref, cnt_ref, hp_ref, w_ref, root_ref, b_ref, o_ref):
    bf = jnp.bfloat16
    hp = jnp.concatenate([hp_ref[0], hp_ref[1]], axis=1).astype(bf)
    agg = jax.lax.dot(hp, root_ref[...].astype(bf),
                      preferred_element_type=jnp.float32) + b_ref[...]
    for r in range(R):
        tot = cnt_ref[0, r, :, 0:1] + cnt_ref[1, r, :, 0:1]
        inv = 1.0 / jnp.maximum(tot, 1.0)
        for c in range(2):
            agg += jax.lax.dot((bkt_ref[c, r] * inv).astype(bf),
                               w_ref[c, r].astype(bf),
                               preferred_element_type=jnp.float32)
    out = jnp.maximum(agg, 0.0)
    o_ref[0] = out[:, :128]
    o_ref[1] = out[:, 128:]


def _tc_wide(buckets, cnt4, hprev, Wstack, root, b):
    return pl.pallas_call(
        _tc_wide_body,
        grid=(_GRID_B,),
        in_specs=[
            pl.BlockSpec((2, R, _BN, 128), lambda i: (0, 0, i, 0)),
            pl.BlockSpec((2, R, _BN, 16), lambda i: (0, 0, i, 0)),
            pl.BlockSpec((2, _BN, 128), lambda i: (0, i, 0)),
            pl.BlockSpec((2, R, 128, 256), lambda i: (0, 0, 0, 0)),
            pl.BlockSpec((256, 256), lambda i: (0, 0)),
            pl.BlockSpec((1, 256), lambda i: (0, 0)),
        ],
        out_specs=pl.BlockSpec((2, _BN, 128), lambda i: (0, i, 0)),
        out_shape=jax.ShapeDtypeStruct((2, N, 128), jnp.float32),
    )(buckets, cnt4, hprev, Wstack, root, b)


def _tc_msg_body(hp_ref, w_ref, o_ref):
    bf = jnp.bfloat16
    hp = jnp.concatenate([hp_ref[0], hp_ref[1]], axis=1).astype(bf)
    for r in range(R):
        o_ref[r] = jax.lax.dot(hp, w_ref[r].astype(bf),
                               preferred_element_type=jnp.float32)


def _tc_msg(hprev, Wpad):
    return pl.pallas_call(
        _tc_msg_body,
        grid=(_GRID_B,),
        in_specs=[
            pl.BlockSpec((2, _BN, 128), lambda i: (0, i, 0)),
            pl.BlockSpec((R, 256, 16), lambda i: (0, 0, 0)),
        ],
        out_specs=pl.BlockSpec((R, _BN, 16), lambda i: (0, i, 0)),
        out_shape=jax.ShapeDtypeStruct((R, N, 16), jnp.float32),
    )(hprev, Wpad)


def _tc_narrow_body(wide_h, bkt_ref, cnt_ref, hp_ref, m_ref, root_ref,
                    b_ref, o_ref):
    if wide_h:
        hp = jnp.concatenate([hp_ref[0], hp_ref[1]], axis=1)
    else:
        hp = hp_ref[...]
    agg = jax.lax.dot(hp, root_ref[...],
                      preferred_element_type=jnp.float32) + b_ref[...]
    for r in range(R):
        tot = cnt_ref[0, r, :, 0:1] + cnt_ref[1, r, :, 0:1]
        inv = 1.0 / jnp.maximum(tot, 1.0)
        agg += jax.lax.dot((bkt_ref[0, r] + bkt_ref[1, r]) * inv, m_ref[r],
                           preferred_element_type=jnp.float32)
    o_ref[...] = jnp.maximum(agg, 0.0)


def _tc_narrow(buckets, cnt4, hprev, mix, rootpad, bpad, wide_h):
    hspec = (pl.BlockSpec((2, _BN, 128), lambda i: (0, i, 0)) if wide_h
             else pl.BlockSpec((_BN, 16), lambda i: (i, 0)))
    rootdim = 256 if wide_h else 16
    return pl.pallas_call(
        functools.partial(_tc_narrow_body, wide_h),
        grid=(_GRID_B,),
        in_specs=[
            pl.BlockSpec((2, R, _BN, 16), lambda i: (0, 0, i, 0)),
            pl.BlockSpec((2, R, _BN, 16), lambda i: (0, 0, i, 0)),
            hspec,
            pl.BlockSpec((R, 16, 16), lambda i: (0, 0, 0)),
            pl.BlockSpec((rootdim, 16), lambda i: (0, 0)),
            pl.BlockSpec((1, 16), lambda i: (0, 0)),
        ],
        out_specs=pl.BlockSpec((_BN, 16), lambda i: (i, 0)),
        out_shape=jax.ShapeDtypeStruct((N, 16), jnp.float32),
    )(buckets, cnt4, hprev, mix, rootpad, bpad)


def _matvec_body(act, v_ref, m_ref, b_ref, o_ref, acc_ref):
    kk = pl.program_id(1)
    nk = pl.num_programs(1)

    @pl.when(kk == 0)
    def _():
        acc_ref[...] = jnp.zeros_like(acc_ref)

    acc_ref[...] += jax.lax.dot(v_ref[...], m_ref[...],
                                preferred_element_type=jnp.float32)

    @pl.when(kk == nk - 1)
    def _():
        o_ref[...] = act(acc_ref[...] + b_ref[...])


def _matvec(v, M, b, act, bn=512, bk=2048):
    K, Nout = M.shape
    out = pl.pallas_call(
        functools.partial(_matvec_body, act),
        grid=(Nout // bn, K // bk),
        in_specs=[
            pl.BlockSpec((1, bk), lambda j, k: (0, k)),
            pl.BlockSpec((bk, bn), lambda j, k: (k, j)),
            pl.BlockSpec((1, bn), lambda j, k: (0, j)),
        ],
        out_specs=pl.BlockSpec((1, bn), lambda j, k: (0, j)),
        out_shape=jax.ShapeDtypeStruct((1, Nout), jnp.float32),
        scratch_shapes=[pltpu.VMEM((1, bn), jnp.float32)],
    )(v.reshape(1, K), M, b.reshape(1, Nout))
    return out.reshape(Nout)


def kernel(x, edge_index, edge_type, W0, root0, b0, W1, root1, b1, W2,
           root2, b2, W3, root3, b3, W4, root4, b4, pw1, pb1, pw2, pb2,
           pw3, pb3):
    src, dst = edge_index[0], edge_index[1]
    key = edge_type * N + dst
    keyrows = key.reshape(E // 128, 128)
    src2 = jnp.concatenate([src, src + N]).reshape(2 * E // 128, 128)
    gidx3 = (edge_type * N + src).reshape(E // 128, 128)
    srcrows = src.reshape(E // 128, 128)
    ones128 = jnp.ones((128, 16), jnp.float32)
    zeros16 = jnp.zeros((NR // NS, 16), jnp.float32)
    zeros64 = jnp.zeros((64, 128), jnp.float32)

    cnt4 = _sc_count(keyrows, ones128, zeros16)

    def pad16(w):
        return jnp.pad(w, [(0, 0)] * (w.ndim - 1) + [(0, 16 - w.shape[-1])])

    h2 = x.reshape(N, 2, 128).transpose(1, 0, 2)
    for (W, root, b) in [(W0, root0, b0), (W1, root1, b1), (W2, root2, b2)]:
        buckets = _sc_bucket_wide(h2.reshape(2 * N, 128), src2, keyrows,
                                  zeros64)
        Wstack = jnp.stack([W[:, :128, :], W[:, 128:, :]])
        h2 = _tc_wide(buckets, cnt4, h2, Wstack, root, b.reshape(1, 256))

    am_tab = _tc_msg(h2, pad16(W3))
    b3kt = _sc_bucket_narrow(am_tab.reshape(R * N, 16), gidx3, keyrows,
                             zeros16)
    sum_r = jnp.broadcast_to(jnp.eye(16, dtype=jnp.float32), (R, 16, 16))
    h4 = _tc_narrow(b3kt, cnt4, h2, sum_r, pad16(root3),
                    pad16(b3).reshape(1, 16), wide_h=True)

    b4kt = _sc_bucket_narrow(h4, srcrows, keyrows, zeros16)
    W4mix = pad16(jnp.pad(W4, ((0, 0), (0, 13), (0, 0))))
    root4pad = pad16(jnp.pad(root4, ((0, 13), (0, 0))))
    h5 = _tc_narrow(b4kt, cnt4, h4, W4mix, root4pad,
                    pad16(b4).reshape(1, 16), wide_h=False)

    flat = h5[:, :3].reshape(-1)
    z = _matvec(flat, pw1, pb1, jnp.tanh, bn=2048)
    z = _matvec(z, pw2, pb2, jnp.tanh, bn=2048)
    z = _matvec(z, pw3, pb3, jax.nn.sigmoid, bn=2048)
    return z
